# baseline (speedup 1.0000x reference)
import functools

import jax
import jax.numpy as jnp
from jax import lax
from jax.experimental import pallas as pl
from jax.experimental.pallas import tpu as pltpu
from jax.experimental.pallas import tpu_sc as plsc

N = 100000
E = 1600000
C = 16
EPS = 1e-6
NEG_SLOPE = 0.2

NC = 2
NS = 16
L = 16

BG = 800
N_PAD = 100096
ROWS_PER_TILE = N_PAD // NS
ZB = 368
BLK_FULL = E // (NS * BG)
BLK_C2_SC0 = 63
BLK_C2_SC1 = 62
C2_SPLIT = BLK_C2_SC0 * NS * BG

BN = 2000


def _sc_pool_body(x3_hbm, edges_hbm, out_hbm,
                  idx_a, idx_b, e2a, e2b, rows_a, rows_b, acc_sh,
                  g_a, g_b, s_a, s_b):
    c = lax.axis_index("c")
    s = lax.axis_index("s")
    row0 = s * ROWS_PER_TILE

    def zero_fill():
        def zb(j, _):
            rows_a[j, :] = jnp.zeros((L,), jnp.float32)
            return 0
        lax.fori_loop(0, ZB, zb, 0)

    def clear_acc():
        def cb(p, _):
            pltpu.sync_copy(rows_a.at[pl.ds(0, ZB)],
                            acc_sh.at[pl.ds(row0 + p * ZB, ZB)])
            return 0
        lax.fori_loop(0, ROWS_PER_TILE // ZB, cb, 0)

    def accumulate(chunk, lo, nblk):
        table = x3_hbm.at[chunk]

        def stage_fire(q, idxv, rowsv, e2v, gsem):
            off = lo + (s * nblk + q) * BG
            pltpu.sync_copy(edges_hbm.at[0, pl.ds(off, BG)], idxv)
            pltpu.sync_copy(edges_hbm.at[1, pl.ds(off, BG)], e2v)
            pltpu.async_copy(table.at[idxv], rowsv, gsem)

        def wait_gather(idxv, rowsv, gsem):
            pltpu.make_async_copy(table.at[idxv], rowsv, gsem).wait()

        def fire_scatter(rowsv, e2v, ssem):
            pltpu.async_copy(rowsv, acc_sh.at[e2v], ssem, add=True)

        def wait_scatter(rowsv, e2v, ssem):
            pltpu.make_async_copy(rowsv, acc_sh.at[e2v], ssem).wait()

        stage_fire(0, idx_a, rows_a, e2a, g_a)

        def body(q, _):
            nxt = q + 1
            q_even = lax.rem(q, 2) == 0
            has_nxt = nxt < nblk

            @pl.when(jnp.logical_and(has_nxt,
                                     jnp.logical_and(q_even, q > 0)))
            def _():
                wait_scatter(rows_b, e2b, s_b)

            @pl.when(jnp.logical_and(has_nxt, jnp.logical_not(q_even)))
            def _():
                wait_scatter(rows_a, e2a, s_a)

            @pl.when(jnp.logical_and(has_nxt, q_even))
            def _():
                stage_fire(nxt, idx_b, rows_b, e2b, g_b)

            @pl.when(jnp.logical_and(has_nxt, jnp.logical_not(q_even)))
            def _():
                stage_fire(nxt, idx_a, rows_a, e2a, g_a)

            @pl.when(q_even)
            def _():
                wait_gather(idx_a, rows_a, g_a)
                fire_scatter(rows_a, e2a, s_a)

            @pl.when(jnp.logical_not(q_even))
            def _():
                wait_gather(idx_b, rows_b, g_b)
                fire_scatter(rows_b, e2b, s_b)
            return 0
        lax.fori_loop(0, nblk, body, 0)
        wait_scatter(rows_a, e2a, s_a)
        wait_scatter(rows_b, e2b, s_b)

    def dump(slot):
        def db(p, _):
            r = row0 + p * ZB
            pltpu.sync_copy(acc_sh.at[pl.ds(r, ZB)],
                            out_hbm.at[slot, pl.ds(r, ZB), :])
            return 0
        lax.fori_loop(0, ROWS_PER_TILE // ZB, db, 0)

    zero_fill()
    clear_acc()
    plsc.subcore_barrier()
    accumulate(c, 0, BLK_FULL)
    plsc.subcore_barrier()
    dump(c)
    zero_fill()
    clear_acc()
    plsc.subcore_barrier()
    accumulate(2, c * C2_SPLIT, BLK_C2_SC0 - c)
    plsc.subcore_barrier()
    dump(2 + c)


def _sc_pool(x3, edges):
    mesh = plsc.VectorSubcoreMesh(core_axis_name="c", subcore_axis_name="s",
                                  num_cores=NC, num_subcores=NS)
    fn = pl.kernel(
        _sc_pool_body,
        out_type=jax.ShapeDtypeStruct((4, N_PAD, C), jnp.float32),
        mesh=mesh,
        scratch_types=[
            pltpu.VMEM((BG,), jnp.int32),
            pltpu.VMEM((BG,), jnp.int32),
            pltpu.VMEM((BG,), jnp.int32),
            pltpu.VMEM((BG,), jnp.int32),
            pltpu.VMEM((BG, C), jnp.float32),
            pltpu.VMEM((BG, C), jnp.float32),
            pltpu.VMEM_SHARED((N_PAD, C), jnp.float32),
            pltpu.SemaphoreType.DMA,
            pltpu.SemaphoreType.DMA,
            pltpu.SemaphoreType.DMA,
            pltpu.SemaphoreType.DMA,
        ],
        compiler_params=pltpu.CompilerParams(use_tc_tiling_on_sc=False),
    )
    return fn(x3, edges)


BNP = 736
NP = N // 8
NP_PAD = N_PAD // 8


def _dense_body(xt_ref, p_ref, wi_ref, wp_ref, wd_ref, b_ref, out_ref):
    wi = wi_ref[:]
    wp = wp_ref[:]
    wd = wd_ref[:]
    b = b_ref[0:1, :]
    hs = []
    ds = []
    dot = jnp.zeros((BNP, 128), jnp.float32)
    dn = jnp.zeros((BNP, 128), jnp.float32)
    for k in range(3):
        xk = xt_ref[k]
        pk = p_ref[k]
        if k == 2:
            pk = pk + p_ref[3]
        h = (jnp.dot(xk, wi, preferred_element_type=jnp.float32)
             + jnp.dot(pk, wp, preferred_element_type=jnp.float32) + b)
        d = jnp.dot(h, wd, preferred_element_type=jnp.float32)
        dot = dot + h * d
        dn = dn + d * d
        hs.append(h)
        ds.append(d)
    coef = jnp.where(dot >= 0, jnp.zeros_like(dot),
                     (1.0 - NEG_SLOPE) * dot / (dn + EPS))
    for k in range(3):
        out_ref[k] = xt_ref[k] + hs[k] - coef * ds[k]


def _dense(xtp, p4p, wi, wp, wd, b8):
    return pl.pallas_call(
        _dense_body,
        grid=(NP_PAD // BNP,),
        in_specs=[
            pl.BlockSpec((3, BNP, 128), lambda i: (0, i, 0)),
            pl.BlockSpec((4, BNP, 128), lambda i: (0, i, 0)),
            pl.BlockSpec((128, 128), lambda i: (0, 0)),
            pl.BlockSpec((128, 128), lambda i: (0, 0)),
            pl.BlockSpec((128, 128), lambda i: (0, 0)),
            pl.BlockSpec((8, 128), lambda i: (0, 0)),
        ],
        out_specs=pl.BlockSpec((3, BNP, 128), lambda i: (0, i, 0)),
        out_shape=jax.ShapeDtypeStruct((3, NP_PAD, 128), jnp.float32),
    )(xtp, p4p, wi, wp, wd, b8)


def kernel(x, edges, W_id, b_id, W_pool, b_pool, W_dir):
    xpad = jnp.pad(x, ((0, N_PAD - N), (0, 0), (0, 0)))
    xt = jnp.transpose(xpad, (1, 0, 2))
    p4 = _sc_pool(xt, edges)
    xtp = xt.reshape(3, NP_PAD, 128)
    p4p = p4.reshape(4, NP_PAD, 128)
    eye8 = jnp.eye(8, dtype=jnp.float32)
    wi = jnp.kron(eye8, W_id.T)
    wp = jnp.kron(eye8, W_pool.T)
    wd = jnp.kron(eye8, W_dir.T)
    b8 = jnp.broadcast_to(jnp.tile(b_id + b_pool, 8), (8, 128))
    out_p = _dense(xtp, p4p, wi, wp, wd, b8)
    out = jnp.transpose(out_p.reshape(3, N_PAD, C)[:, :N], (1, 0, 2))
    return (out, edges)


# speedup vs baseline: 1.2395x; 1.2395x over previous
import functools

import jax
import jax.numpy as jnp
from jax import lax
from jax.experimental import pallas as pl
from jax.experimental.pallas import tpu as pltpu
from jax.experimental.pallas import tpu_sc as plsc

N = 100000
E = 1600000
C = 16
EPS = 1e-6
NEG_SLOPE = 0.2

NC = 2
NS = 16
L = 16

BG = 800
N_PAD = 100096
ROWS_PER_TILE = N_PAD // NS
ZB = 368
BLK_FULL = E // (NS * BG)
BLK_C2_SC0 = 63
BLK_C2_SC1 = 62
C2_SPLIT = BLK_C2_SC0 * NS * BG

BN = 2000


def _sc_pool_body(x3_hbm, edges_hbm, out_hbm,
                  idx_a, idx_b, e2a, e2b, rows_a, rows_b, acc_sh,
                  g_a, g_b, s_a, s_b):
    c = lax.axis_index("c")
    s = lax.axis_index("s")
    row0 = s * ROWS_PER_TILE

    def zero_fill():
        def zb(j, _):
            rows_a[j, :] = jnp.zeros((L,), jnp.float32)
            return 0
        lax.fori_loop(0, ZB, zb, 0)

    def clear_acc():
        def cb(p, _):
            pltpu.sync_copy(rows_a.at[pl.ds(0, ZB)],
                            acc_sh.at[pl.ds(row0 + p * ZB, ZB)])
            return 0
        lax.fori_loop(0, ROWS_PER_TILE // ZB, cb, 0)

    def accumulate(chunk, lo, nblk):
        table = x3_hbm.at[chunk]

        def stage_fire(q, idxv, rowsv, e2v, gsem):
            off = lo + (s * nblk + q) * BG
            pltpu.sync_copy(edges_hbm.at[0, pl.ds(off, BG)], idxv)
            pltpu.sync_copy(edges_hbm.at[1, pl.ds(off, BG)], e2v)
            pltpu.async_copy(table.at[idxv], rowsv, gsem)

        def wait_gather(idxv, rowsv, gsem):
            pltpu.make_async_copy(table.at[idxv], rowsv, gsem).wait()

        def fire_scatter(rowsv, e2v, ssem):
            pltpu.async_copy(rowsv, acc_sh.at[e2v], ssem, add=True)

        def wait_scatter(rowsv, e2v, ssem):
            pltpu.make_async_copy(rowsv, acc_sh.at[e2v], ssem).wait()

        stage_fire(0, idx_a, rows_a, e2a, g_a)

        def body(q, _):
            nxt = q + 1
            q_even = lax.rem(q, 2) == 0
            has_nxt = nxt < nblk

            @pl.when(jnp.logical_and(has_nxt,
                                     jnp.logical_and(q_even, q > 0)))
            def _():
                wait_scatter(rows_b, e2b, s_b)

            @pl.when(jnp.logical_and(has_nxt, jnp.logical_not(q_even)))
            def _():
                wait_scatter(rows_a, e2a, s_a)

            @pl.when(jnp.logical_and(has_nxt, q_even))
            def _():
                stage_fire(nxt, idx_b, rows_b, e2b, g_b)

            @pl.when(jnp.logical_and(has_nxt, jnp.logical_not(q_even)))
            def _():
                stage_fire(nxt, idx_a, rows_a, e2a, g_a)

            @pl.when(q_even)
            def _():
                wait_gather(idx_a, rows_a, g_a)
                fire_scatter(rows_a, e2a, s_a)

            @pl.when(jnp.logical_not(q_even))
            def _():
                wait_gather(idx_b, rows_b, g_b)
                fire_scatter(rows_b, e2b, s_b)
            return 0
        lax.fori_loop(0, nblk, body, 0)
        wait_scatter(rows_a, e2a, s_a)
        wait_scatter(rows_b, e2b, s_b)

    def dump(slot):
        def db(p, _):
            r = row0 + p * ZB
            pltpu.sync_copy(acc_sh.at[pl.ds(r, ZB)],
                            out_hbm.at[slot, pl.ds(r, ZB), :])
            return 0
        lax.fori_loop(0, ROWS_PER_TILE // ZB, db, 0)

    zero_fill()
    clear_acc()
    plsc.subcore_barrier()
    accumulate(c, 0, BLK_FULL)
    plsc.subcore_barrier()
    dump(c)
    zero_fill()
    clear_acc()
    plsc.subcore_barrier()
    accumulate(2, c * C2_SPLIT, BLK_C2_SC0 - c)
    plsc.subcore_barrier()
    dump(2 + c)


def _sc_pool(x3, edges):
    mesh = plsc.VectorSubcoreMesh(core_axis_name="c", subcore_axis_name="s",
                                  num_cores=NC, num_subcores=NS)
    fn = pl.kernel(
        _sc_pool_body,
        out_type=jax.ShapeDtypeStruct((4, N_PAD, C), jnp.float32),
        mesh=mesh,
        scratch_types=[
            pltpu.VMEM((BG,), jnp.int32),
            pltpu.VMEM((BG,), jnp.int32),
            pltpu.VMEM((BG,), jnp.int32),
            pltpu.VMEM((BG,), jnp.int32),
            pltpu.VMEM((BG, C), jnp.float32),
            pltpu.VMEM((BG, C), jnp.float32),
            pltpu.VMEM_SHARED((N_PAD, C), jnp.float32),
            pltpu.SemaphoreType.DMA,
            pltpu.SemaphoreType.DMA,
            pltpu.SemaphoreType.DMA,
            pltpu.SemaphoreType.DMA,
        ],
        compiler_params=pltpu.CompilerParams(use_tc_tiling_on_sc=False),
    )
    return fn(x3, edges)


BNP = 736
NP = N // 8
NP_PAD = N_PAD // 8


def _dense_body(xt_ref, p_ref, wi_ref, wp_ref, wd_ref, b_ref, out_ref):
    wi = wi_ref[:]
    wp = wp_ref[:]
    wd = wd_ref[:]
    b = b_ref[0:1, :]
    hs = []
    ds = []
    dot = jnp.zeros((BNP, 128), jnp.float32)
    dn = jnp.zeros((BNP, 128), jnp.float32)
    for k in range(3):
        xk = xt_ref[k]
        pk = p_ref[k]
        if k == 2:
            pk = pk + p_ref[3]
        h = (jnp.dot(xk, wi, preferred_element_type=jnp.float32)
             + jnp.dot(pk, wp, preferred_element_type=jnp.float32) + b)
        d = jnp.dot(h, wd, preferred_element_type=jnp.float32)
        dot = dot + h * d
        dn = dn + d * d
        hs.append(h)
        ds.append(d)
    coef = jnp.where(dot >= 0, jnp.zeros_like(dot),
                     (1.0 - NEG_SLOPE) * dot / (dn + EPS))
    for k in range(3):
        out_ref[k] = xt_ref[k] + hs[k] - coef * ds[k]


def _dense(xtp, p4p, wi, wp, wd, b8):
    return pl.pallas_call(
        _dense_body,
        grid=((NP + BNP - 1) // BNP,),
        in_specs=[
            pl.BlockSpec((3, BNP, 128), lambda i: (0, i, 0)),
            pl.BlockSpec((4, BNP, 128), lambda i: (0, i, 0)),
            pl.BlockSpec((128, 128), lambda i: (0, 0)),
            pl.BlockSpec((128, 128), lambda i: (0, 0)),
            pl.BlockSpec((128, 128), lambda i: (0, 0)),
            pl.BlockSpec((8, 128), lambda i: (0, 0)),
        ],
        out_specs=pl.BlockSpec((3, BNP, 128), lambda i: (0, i, 0)),
        out_shape=jax.ShapeDtypeStruct((3, NP, 128), jnp.float32),
    )(xtp, p4p, wi, wp, wd, b8)


def kernel(x, edges, W_id, b_id, W_pool, b_pool, W_dir):
    xt = jnp.transpose(x, (1, 0, 2))
    p4 = _sc_pool(xt, edges)
    xtp = xt.reshape(3, NP, 128)
    p4p = p4.reshape(4, NP_PAD, 128)
    eye8 = jnp.eye(8, dtype=jnp.float32)
    wi = jnp.kron(eye8, W_id.T)
    wp = jnp.kron(eye8, W_pool.T)
    wd = jnp.kron(eye8, W_dir.T)
    b8 = jnp.broadcast_to(jnp.tile(b_id + b_pool, 8), (8, 128))
    out_p = _dense(xtp, p4p, wi, wp, wd, b8)
    out = jnp.transpose(out_p.reshape(3, N, C), (1, 0, 2))
    return (out, edges)
